# baseline (speedup 1.0000x reference)
import functools

import jax
import jax.numpy as jnp
from jax import lax
from jax.experimental import pallas as pl
from jax.experimental.pallas import tpu as pltpu
from jax.experimental.pallas import tpu_sc as plsc

_BETA = 0.25
_NE = 8192
_DIM = 256
_BR = 256
_NROWS = 9216
_NBLK = _NROWS // _BR

_NW = 32
_BPW = _NROWS // _NW
_CH = 96
_NCH = _BPW // _CH


_RG = 32
_CW = 128
_NG = _NE // _CW


def _argmin_body(z_ref, et2_ref, en_ref, idx_ref, dsum_ref, s2_ref):
    i = pl.program_id(0)
    s2_ref[...] = jnp.dot(z_ref[...], et2_ref[...],
                          preferred_element_type=jnp.float32)

    @pl.when(i == 0)
    def _init():
        dsum_ref[0, 0] = 0.0

    def row_group(rg, dacc):
        zrows = z_ref[pl.ds(rg * _RG, _RG), :]
        zn = jnp.sum(zrows * zrows, axis=1, keepdims=True)
        sval = (zn + en_ref[0:1, 0:_CW]) + s2_ref[pl.ds(rg * _RG, _RG), 0:_CW]
        sg = jnp.zeros((_RG, _CW), jnp.float32)
        for g in range(1, _NG):
            d = ((zn + en_ref[0:1, g * _CW:(g + 1) * _CW])
                 + s2_ref[pl.ds(rg * _RG, _RG), g * _CW:(g + 1) * _CW])
            cmp = d < sval
            sval = jnp.minimum(sval, d)
            sg = jnp.where(cmp, jnp.float32(g), sg)
        m = jnp.min(sval, axis=1, keepdims=True)
        lane = lax.broadcasted_iota(jnp.int32, (_RG, _CW), 1).astype(jnp.float32)
        jfull = sg * jnp.float32(_CW) + lane
        idxf = jnp.min(jnp.where(sval == m, jfull, jnp.float32(2.0 * _NE)),
                       axis=1, keepdims=True)
        idx_ref[pl.ds(i * _BR + rg * _RG, _RG), :] = idxf.astype(jnp.int32)
        return dacc + jnp.sum(m)

    dsum_ref[0, 0] += lax.fori_loop(0, _BR // _RG, row_group,
                                    jnp.float32(0.0))


_dist_argmin = pl.pallas_call(
    _argmin_body,
    grid=(_NBLK,),
    in_specs=[
        pl.BlockSpec((_BR, _DIM), lambda i: (i, 0)),
        pl.BlockSpec((_DIM, _NE), lambda i: (0, 0)),
        pl.BlockSpec((1, _NE), lambda i: (0, 0)),
    ],
    out_specs=[
        pl.BlockSpec((_NROWS, 1), lambda i: (0, 0)),
        pl.BlockSpec(memory_space=pltpu.SMEM),
    ],
    out_shape=[
        jax.ShapeDtypeStruct((_NROWS, 1), jnp.int32),
        jax.ShapeDtypeStruct((1, 1), jnp.float32),
    ],
    scratch_shapes=[pltpu.VMEM((_BR, _NE), jnp.float32)],
)


_sc_mesh = plsc.VectorSubcoreMesh(core_axis_name="c", subcore_axis_name="s")


@functools.partial(
    pl.kernel,
    mesh=_sc_mesh,
    out_type=jax.ShapeDtypeStruct((_NROWS, _DIM), jnp.float32),
    scratch_types=[
        pltpu.VMEM((_NCH, _CH), jnp.int32),
        pltpu.VMEM((_BPW, _DIM), jnp.float32),
        pltpu.SemaphoreType.DMA,
    ],
)
def _gather_rows(table_hbm, idx_hbm, out_hbm, idx_v, rows_v, sem):
    wid = lax.axis_index("s") * 2 + lax.axis_index("c")
    base = wid * _BPW
    for j in range(_NCH):
        pltpu.sync_copy(idx_hbm.at[pl.ds(base + j * _CH, _CH)], idx_v.at[j])
    copies = [
        pltpu.async_copy(table_hbm.at[idx_v.at[j]],
                         rows_v.at[pl.ds(j * _CH, _CH)], sem)
        for j in range(_NCH)
    ]
    for c in copies:
        c.wait()
    pltpu.sync_copy(rows_v, out_hbm.at[pl.ds(base, _BPW)])


def kernel(z, embedding):
    z_flat = z.reshape(-1, _DIM)
    et2 = embedding.T * jnp.float32(-2.0)
    en = jnp.sum(embedding ** 2, axis=1).reshape(1, _NE)
    idx2d, dsum = _dist_argmin(z_flat, et2, en)
    idx = idx2d.reshape(-1)
    zq_flat = _gather_rows(embedding, idx)
    z_q = zq_flat.reshape(z.shape)
    z_q_out = jnp.transpose(z_q, (0, 3, 1, 2))
    diff = dsum[0, 0] * (_BETA / z.size)
    perplexity = jnp.float32(1.0)
    return (z_q_out, diff, idx, perplexity)


# speedup vs baseline: 1.2163x; 1.2163x over previous
import functools

import jax
import jax.numpy as jnp
from jax import lax
from jax.experimental import pallas as pl
from jax.experimental.pallas import tpu as pltpu
from jax.experimental.pallas import tpu_sc as plsc

_BETA = 0.25
_NE = 8192
_DIM = 256
_BR = 256
_NROWS = 9216
_NBLK = _NROWS // _BR

_NW = 32
_BPW = _NROWS // _NW
_CH = 96
_NCH = _BPW // _CH


_RG = 32
_CW = 128
_NG = _NE // _CW


def _argmin_body(z_ref, et2_ref, en_ref, idx_ref, dsum_ref, d_ref):
    i = pl.program_id(0)
    z_blk = z_ref[...]
    s2 = jnp.dot(z_blk, et2_ref[...],
                 preferred_element_type=jnp.float32)
    zn = jnp.sum(z_blk * z_blk, axis=1, keepdims=True)
    d_ref[...] = (zn + en_ref[...]) + s2

    @pl.when(i == 0)
    def _init():
        dsum_ref[0, 0] = 0.0

    def row_group(rg, dacc):
        base = pl.multiple_of(rg * _RG, _RG)
        sval = d_ref[pl.ds(base, _RG), 0:_CW]
        sg = jnp.zeros((_RG, _CW), jnp.float32)
        for g in range(1, _NG):
            d = d_ref[pl.ds(base, _RG), g * _CW:(g + 1) * _CW]
            cmp = d < sval
            sval = jnp.minimum(sval, d)
            sg = jnp.where(cmp, jnp.float32(g), sg)
        m = jnp.min(sval, axis=1, keepdims=True)
        lane = lax.broadcasted_iota(jnp.int32, (_RG, _CW), 1).astype(jnp.float32)
        jfull = sg * jnp.float32(_CW) + lane
        idxf = jnp.min(jnp.where(sval == m, jfull, jnp.float32(2.0 * _NE)),
                       axis=1, keepdims=True)
        idx_ref[pl.ds(i * _BR + base, _RG), :] = idxf.astype(jnp.int32)
        return dacc + jnp.sum(m)

    dsum_ref[0, 0] += lax.fori_loop(0, _BR // _RG, row_group,
                                    jnp.float32(0.0))


_dist_argmin = pl.pallas_call(
    _argmin_body,
    grid=(_NBLK,),
    in_specs=[
        pl.BlockSpec((_BR, _DIM), lambda i: (i, 0)),
        pl.BlockSpec((_DIM, _NE), lambda i: (0, 0)),
        pl.BlockSpec((1, _NE), lambda i: (0, 0)),
    ],
    out_specs=[
        pl.BlockSpec((_NROWS, 1), lambda i: (0, 0)),
        pl.BlockSpec(memory_space=pltpu.SMEM),
    ],
    out_shape=[
        jax.ShapeDtypeStruct((_NROWS, 1), jnp.int32),
        jax.ShapeDtypeStruct((1, 1), jnp.float32),
    ],
    scratch_shapes=[pltpu.VMEM((_BR, _NE), jnp.float32)],
)


_sc_mesh = plsc.VectorSubcoreMesh(core_axis_name="c", subcore_axis_name="s")


@functools.partial(
    pl.kernel,
    mesh=_sc_mesh,
    out_type=jax.ShapeDtypeStruct((_NROWS, _DIM), jnp.float32),
    scratch_types=[
        pltpu.VMEM((_NCH, _CH), jnp.int32),
        pltpu.VMEM((_BPW, _DIM), jnp.float32),
        pltpu.SemaphoreType.DMA,
    ],
)
def _gather_rows(table_hbm, idx_hbm, out_hbm, idx_v, rows_v, sem):
    wid = lax.axis_index("s") * 2 + lax.axis_index("c")
    base = wid * _BPW
    for j in range(_NCH):
        pltpu.sync_copy(idx_hbm.at[pl.ds(base + j * _CH, _CH)], idx_v.at[j])
    copies = [
        pltpu.async_copy(table_hbm.at[idx_v.at[j]],
                         rows_v.at[pl.ds(j * _CH, _CH)], sem)
        for j in range(_NCH)
    ]
    for c in copies:
        c.wait()
    pltpu.sync_copy(rows_v, out_hbm.at[pl.ds(base, _BPW)])


def kernel(z, embedding):
    z_flat = z.reshape(-1, _DIM)
    et2 = embedding.T * jnp.float32(-2.0)
    en = jnp.sum(embedding ** 2, axis=1).reshape(1, _NE)
    idx2d, dsum = _dist_argmin(z_flat, et2, en)
    idx = idx2d.reshape(-1)
    zq_flat = _gather_rows(embedding, idx)
    z_q = zq_flat.reshape(z.shape)
    z_q_out = jnp.transpose(z_q, (0, 3, 1, 2))
    diff = dsum[0, 0] * (_BETA / z.size)
    perplexity = jnp.float32(1.0)
    return (z_q_out, diff, idx, perplexity)


# speedup vs baseline: 1.5949x; 1.3112x over previous
import functools

import jax
import jax.numpy as jnp
from jax import lax
from jax.experimental import pallas as pl
from jax.experimental.pallas import tpu as pltpu
from jax.experimental.pallas import tpu_sc as plsc

_BETA = 0.25
_NE = 8192
_DIM = 256
_BR = 256
_NROWS = 9216
_NBLK = _NROWS // _BR

_NW = 32
_BPW = _NROWS // _NW
_CH = 96
_NCH = _BPW // _CH


def _argmin_body(z_ref, et_ref, en_ref, idx_ref, dsum_ref):
    i = pl.program_id(0)
    z_blk = z_ref[...]
    s = jnp.dot(z_blk, et_ref[...],
                preferred_element_type=jnp.float32)
    zn = jnp.sum(z_blk * z_blk, axis=1, keepdims=True)
    d = (zn + en_ref[...]) - 2.0 * s
    m = jnp.min(d, axis=1, keepdims=True)
    ji = lax.broadcasted_iota(jnp.int32, d.shape, 1)
    idx = jnp.min(jnp.where(d == m, ji, _NE), axis=1)
    idx_ref[pl.ds(i, 1), :] = idx.reshape(1, _BR)

    @pl.when(i == 0)
    def _init():
        dsum_ref[0, 0] = 0.0

    dsum_ref[0, 0] += jnp.sum(m[:, 0])


_dist_argmin = pl.pallas_call(
    _argmin_body,
    grid=(_NBLK,),
    in_specs=[
        pl.BlockSpec((_BR, _DIM), lambda i: (i, 0)),
        pl.BlockSpec((_DIM, _NE), lambda i: (0, 0)),
        pl.BlockSpec((1, _NE), lambda i: (0, 0)),
    ],
    out_specs=[
        pl.BlockSpec((_NBLK, _BR), lambda i: (0, 0)),
        pl.BlockSpec(memory_space=pltpu.SMEM),
    ],
    out_shape=[
        jax.ShapeDtypeStruct((_NBLK, _BR), jnp.int32),
        jax.ShapeDtypeStruct((1, 1), jnp.float32),
    ],
)


_sc_mesh = plsc.VectorSubcoreMesh(core_axis_name="c", subcore_axis_name="s")


@functools.partial(
    pl.kernel,
    mesh=_sc_mesh,
    out_type=jax.ShapeDtypeStruct((_NROWS, _DIM), jnp.float32),
    scratch_types=[
        pltpu.VMEM((_NCH, _CH), jnp.int32),
        pltpu.VMEM((_BPW, _DIM), jnp.float32),
        pltpu.SemaphoreType.DMA,
    ],
)
def _gather_rows(table_hbm, idx_hbm, out_hbm, idx_v, rows_v, sem):
    wid = lax.axis_index("s") * 2 + lax.axis_index("c")
    base = wid * _BPW
    for j in range(_NCH):
        pltpu.sync_copy(idx_hbm.at[pl.ds(base + j * _CH, _CH)], idx_v.at[j])
    copies = [
        pltpu.async_copy(table_hbm.at[idx_v.at[j]],
                         rows_v.at[pl.ds(j * _CH, _CH)], sem)
        for j in range(_NCH)
    ]
    for c in copies:
        c.wait()
    pltpu.sync_copy(rows_v, out_hbm.at[pl.ds(base, _BPW)])


def kernel(z, embedding):
    z_flat = z.reshape(-1, _DIM)
    e_t = embedding.T
    en = jnp.sum(embedding ** 2, axis=1).reshape(1, _NE)
    idx2d, dsum = _dist_argmin(z_flat, e_t, en)
    idx = idx2d.reshape(-1)
    zq_flat = _gather_rows(embedding, idx)
    z_q = zq_flat.reshape(z.shape)
    z_q_out = jnp.transpose(z_q, (0, 3, 1, 2))
    diff = dsum[0, 0] * (_BETA / z.size)
    perplexity = jnp.float32(1.0)
    return (z_q_out, diff, idx, perplexity)


# speedup vs baseline: 1.6456x; 1.0318x over previous
import functools

import jax
import jax.numpy as jnp
from jax import lax
from jax.experimental import pallas as pl
from jax.experimental.pallas import tpu as pltpu
from jax.experimental.pallas import tpu_sc as plsc

_BETA = 0.25
_NE = 8192
_DIM = 256
_BR = 512
_NROWS = 9216
_NBLK = _NROWS // _BR

_NW = 32
_BPW = _NROWS // _NW
_CH = 96
_NCH = _BPW // _CH


def _argmin_body(z_ref, et_ref, en_ref, idx_ref, dsum_ref):
    i = pl.program_id(0)
    z_blk = z_ref[...]
    s = jnp.dot(z_blk, et_ref[...],
                preferred_element_type=jnp.float32)
    zn = jnp.sum(z_blk * z_blk, axis=1, keepdims=True)
    d = (zn + en_ref[...]) - 2.0 * s
    m = jnp.min(d, axis=1, keepdims=True)
    ji = lax.broadcasted_iota(jnp.int32, d.shape, 1)
    idx = jnp.min(jnp.where(d == m, ji, _NE), axis=1)
    idx_ref[pl.ds(i, 1), :] = idx.reshape(1, _BR)

    @pl.when(i == 0)
    def _init():
        dsum_ref[0, 0] = 0.0

    dsum_ref[0, 0] += jnp.sum(m[:, 0])


_dist_argmin = pl.pallas_call(
    _argmin_body,
    grid=(_NBLK,),
    in_specs=[
        pl.BlockSpec((_BR, _DIM), lambda i: (i, 0)),
        pl.BlockSpec((_DIM, _NE), lambda i: (0, 0)),
        pl.BlockSpec((1, _NE), lambda i: (0, 0)),
    ],
    out_specs=[
        pl.BlockSpec((_NBLK, _BR), lambda i: (0, 0)),
        pl.BlockSpec(memory_space=pltpu.SMEM),
    ],
    out_shape=[
        jax.ShapeDtypeStruct((_NBLK, _BR), jnp.int32),
        jax.ShapeDtypeStruct((1, 1), jnp.float32),
    ],
)


_sc_mesh = plsc.VectorSubcoreMesh(core_axis_name="c", subcore_axis_name="s")


@functools.partial(
    pl.kernel,
    mesh=_sc_mesh,
    out_type=jax.ShapeDtypeStruct((_NROWS, _DIM), jnp.float32),
    scratch_types=[
        pltpu.VMEM((_NCH, _CH), jnp.int32),
        pltpu.VMEM((_BPW, _DIM), jnp.float32),
        pltpu.SemaphoreType.DMA,
    ],
)
def _gather_rows(table_hbm, idx_hbm, out_hbm, idx_v, rows_v, sem):
    wid = lax.axis_index("s") * 2 + lax.axis_index("c")
    base = wid * _BPW
    for j in range(_NCH):
        pltpu.sync_copy(idx_hbm.at[pl.ds(base + j * _CH, _CH)], idx_v.at[j])
    copies = [
        pltpu.async_copy(table_hbm.at[idx_v.at[j]],
                         rows_v.at[pl.ds(j * _CH, _CH)], sem)
        for j in range(_NCH)
    ]
    for c in copies:
        c.wait()
    pltpu.sync_copy(rows_v, out_hbm.at[pl.ds(base, _BPW)])


def kernel(z, embedding):
    z_flat = z.reshape(-1, _DIM)
    e_t = embedding.T
    en = jnp.sum(embedding ** 2, axis=1).reshape(1, _NE)
    idx2d, dsum = _dist_argmin(z_flat, e_t, en)
    idx = idx2d.reshape(-1)
    zq_flat = _gather_rows(embedding, idx)
    z_q = zq_flat.reshape(z.shape)
    z_q_out = jnp.transpose(z_q, (0, 3, 1, 2))
    diff = dsum[0, 0] * (_BETA / z.size)
    perplexity = jnp.float32(1.0)
    return (z_q_out, diff, idx, perplexity)
